# baseline (speedup 1.0000x reference)
import functools

import jax
import jax.numpy as jnp
from jax import lax
from jax.experimental import pallas as pl
from jax.experimental.pallas import tpu as pltpu
from jax.experimental.pallas import tpu_sc as plsc

N_NODES = 10000
N_EDGES = 160000
D_FEAT = 256

_NUM_CORES = 2
_NUM_SUBCORES = 16
_NW = _NUM_CORES * _NUM_SUBCORES
_B_PER_W = N_EDGES // _NW
_CHUNK = 40
_NCPG = 5
_GRP = _NCPG * _CHUNK
_NSLOT = 2
_NGRP = _B_PER_W // _GRP
_NPAIR = _NGRP // _NSLOT

_mesh = plsc.VectorSubcoreMesh(core_axis_name="c", subcore_axis_name="s")


@functools.partial(
    pl.kernel,
    mesh=_mesh,
    out_type=jax.ShapeDtypeStruct((N_EDGES, D_FEAT), jnp.float32),
    scratch_types=(
        [pltpu.VMEM((_B_PER_W,), jnp.int32)]
        + [pltpu.VMEM((_GRP, D_FEAT), jnp.float32) for _ in range(_NSLOT)]
        + [pltpu.SemaphoreType.DMA for _ in range(_NSLOT * _NCPG + _NSLOT)]
    ),
)
def _gather_rows(idx_hbm, x_hbm, out_hbm, idx_v, *bufs_and_sems):
    grp = bufs_and_sems[:_NSLOT]
    sem_g = bufs_and_sems[_NSLOT:_NSLOT + _NSLOT * _NCPG]
    sem_s = bufs_and_sems[_NSLOT + _NSLOT * _NCPG:]
    wid = lax.axis_index("s") * _NUM_CORES + lax.axis_index("c")
    base = wid * _B_PER_W
    pltpu.sync_copy(idx_hbm.at[pl.ds(base, _B_PER_W)], idx_v)

    def do_group(goff, p, drain):
        if drain is True:
            pltpu.make_async_copy(
                grp[p], out_hbm.at[pl.ds(base, _GRP)], sem_s[p]
            ).wait()
        elif drain is not None:
            @pl.when(drain)
            def _():
                pltpu.make_async_copy(
                    grp[p], out_hbm.at[pl.ds(base, _GRP)], sem_s[p]
                ).wait()
        for b in range(_NCPG):
            pltpu.async_copy(
                x_hbm.at[idx_v.at[pl.ds(goff + b * _CHUNK, _CHUNK)]],
                grp[p].at[pl.ds(b * _CHUNK, _CHUNK)],
                sem_g[p * _NCPG + b],
            )
        for b in range(_NCPG):
            pltpu.make_async_copy(
                x_hbm.at[idx_v.at[pl.ds(goff + b * _CHUNK, _CHUNK)]],
                grp[p].at[pl.ds(b * _CHUNK, _CHUNK)],
                sem_g[p * _NCPG + b],
            ).wait()
        pltpu.async_copy(grp[p], out_hbm.at[pl.ds(base + goff, _GRP)],
                         sem_s[p])

    def body(i, carry):
        for p in range(_NSLOT):
            do_group((_NSLOT * i + p) * _GRP, p, i > 0)
        return carry

    lax.fori_loop(0, _NPAIR, body, 0)
    for g in range(_NSLOT * _NPAIR, _NGRP):
        do_group(g * _GRP, g % _NSLOT, True)
    for p in range(_NSLOT):
        pltpu.make_async_copy(
            grp[p], out_hbm.at[pl.ds(base, _GRP)], sem_s[p]
        ).wait()


def kernel(edge_index, x):
    idx = edge_index[0].astype(jnp.int32)
    return _gather_rows(idx, x)
